# baseline (speedup 1.0000x reference)
import functools

import jax
import jax.numpy as jnp
from jax import lax
from jax.experimental import pallas as pl
from jax.experimental.pallas import tpu as pltpu
from jax.experimental.pallas import tpu_sc as plsc

N = 10000
E = 160000
C = 256
H = 128
MID = 512

KC = 125
IDX_ROWS = E // KC
TPR = IDX_ROWS // 16
INNER = 8

def _deg_kernel_body(dst2, deg_out, dstv, onesv, stagev, accd):
    c = lax.axis_index("c")
    s = lax.axis_index("s")

    for i in range(8):
        onesv[pl.ds(i * 16, 16)] = jnp.ones((16,), jnp.float32)

    @pl.when(s < 15)
    def _():
        for i in range(5):
            pltpu.sync_copy(onesv, accd.at[pl.ds(s * 640 + i * 128, 128)])

    @pl.when(s == 15)
    def _():
        for i in range(3):
            pltpu.sync_copy(onesv, accd.at[pl.ds(9600 + i * 128, 128)])
        pltpu.sync_copy(onesv.at[pl.ds(0, 16)], accd.at[pl.ds(9984, 16)])

    plsc.subcore_barrier()

    base = (c * 16 + s) * (TPR // 2)

    def outer(j, carry):
        pltpu.sync_copy(dst2.at[pl.ds(base + j * INNER, INNER)], dstv)
        for t in range(INNER):
            pltpu.sync_copy(onesv.at[pl.ds(0, KC)],
                            accd.at[dstv.at[t]], add=True)
        return carry

    lax.fori_loop(0, TPR // 2 // INNER, outer, 0)

    plsc.subcore_barrier()

    @pl.when(s < 15)
    def _():
        for i in range(5):
            pltpu.sync_copy(accd.at[pl.ds(s * 640 + i * 128, 128)], stagev)
            pltpu.sync_copy(
                stagev, deg_out.at[pl.ds(c * N + s * 640 + i * 128, 128)])

    @pl.when(s == 15)
    def _():
        for i in range(3):
            pltpu.sync_copy(accd.at[pl.ds(9600 + i * 128, 128)], stagev)
            pltpu.sync_copy(stagev,
                            deg_out.at[pl.ds(c * N + 9600 + i * 128, 128)])
        pltpu.sync_copy(accd.at[pl.ds(9984, 16)], stagev.at[pl.ds(0, 16)])
        pltpu.sync_copy(stagev.at[pl.ds(0, 16)],
                        deg_out.at[pl.ds(c * N + 9984, 16)])


def _agg_kernel_body(xs_lo, xs_hi, src2, dst2, agg_lo, agg_hi,
                     srcv, dstv, rows_a, rows_b, acc,
                     sem_a, sem_b, ssem_a, ssem_b):
    c = lax.axis_index("c")
    s = lax.axis_index("s")
    stage = rows_a

    def run(xs, agg):
        @pl.when(s < 15)
        def _():
            for i in range(5):
                r0 = s * 640 + i * 128
                pltpu.sync_copy(xs.at[pl.ds(r0, 128)], stage)
                pltpu.sync_copy(stage, acc.at[pl.ds(r0, 128)])

        @pl.when(s == 15)
        def _():
            for i in range(3):
                r0 = 9600 + i * 128
                pltpu.sync_copy(xs.at[pl.ds(r0, 128)], stage)
                pltpu.sync_copy(stage, acc.at[pl.ds(r0, 128)])
            pltpu.sync_copy(xs.at[pl.ds(9984, 16)], stage.at[pl.ds(0, 16)])
            pltpu.sync_copy(stage.at[pl.ds(0, 16)], acc.at[pl.ds(9984, 16)])

        plsc.subcore_barrier()
        base = s * TPR
        bufs = (rows_a, rows_b)
        sems = (sem_a, sem_b)
        HALF = TPR // 2

        ssems = (ssem_a, ssem_b)

        def gather(t, b):
            return pltpu.async_copy(xs.at[srcv.at[t]],
                                    bufs[b].at[pl.ds(0, KC)], sems[b])

        def scatter(t, b):
            return pltpu.async_copy(bufs[b].at[pl.ds(0, KC)],
                                    acc.at[dstv.at[t]], ssems[b], add=True)

        for half in range(2):
            r0 = base + half * HALF
            pltpu.sync_copy(src2.at[pl.ds(r0, HALF)], srcv)
            pltpu.sync_copy(dst2.at[pl.ds(r0, HALF)], dstv)
            ga = gather(0, 0)
            gb = gather(1, 1)

            def body(j, carry):
                t0 = j * 2
                ga.wait()
                sa = scatter(t0, 0)
                gb.wait()
                sb = scatter(t0 + 1, 1)
                sa.wait()

                @pl.when(j < HALF // 2 - 1)
                def _():
                    gather(t0 + 2, 0)

                sb.wait()

                @pl.when(j < HALF // 2 - 1)
                def _():
                    gather(t0 + 3, 1)

                return carry

            lax.fori_loop(0, HALF // 2, body, 0)

        plsc.subcore_barrier()

        @pl.when(s < 15)
        def _():
            for i in range(5):
                r0 = s * 640 + i * 128
                pltpu.sync_copy(acc.at[pl.ds(r0, 128)], stage)
                pltpu.sync_copy(stage, agg.at[pl.ds(r0, 128)])

        @pl.when(s == 15)
        def _():
            for i in range(3):
                r0 = 9600 + i * 128
                pltpu.sync_copy(acc.at[pl.ds(r0, 128)], stage)
                pltpu.sync_copy(stage, agg.at[pl.ds(r0, 128)])
            pltpu.sync_copy(acc.at[pl.ds(9984, 16)], stage.at[pl.ds(0, 16)])
            pltpu.sync_copy(stage.at[pl.ds(0, 16)], agg.at[pl.ds(9984, 16)])

    @pl.when(c == 0)
    def _():
        run(xs_lo, agg_lo)

    @pl.when(c == 1)
    def _():
        run(xs_hi, agg_hi)


@functools.cache
def _sc_kernels():
    mesh = plsc.VectorSubcoreMesh(core_axis_name="c", subcore_axis_name="s",
                                  num_cores=2, num_subcores=16)
    deg_kernel = pl.kernel(
        _deg_kernel_body,
        out_type=jax.ShapeDtypeStruct((2 * N,), jnp.float32),
        mesh=mesh,
        scratch_types=[
            pltpu.VMEM((INNER, KC), jnp.int32),
            pltpu.VMEM((128,), jnp.float32),
            pltpu.VMEM((128,), jnp.float32),
            pltpu.VMEM_SHARED((N,), jnp.float32),
        ],
    )
    agg_kernel = pl.kernel(
        _agg_kernel_body,
        out_type=[jax.ShapeDtypeStruct((N, H), jnp.float32),
                  jax.ShapeDtypeStruct((N, H), jnp.float32)],
        mesh=mesh,
        scratch_types=[
            pltpu.VMEM((TPR // 2, KC), jnp.int32),
            pltpu.VMEM((TPR // 2, KC), jnp.int32),
            pltpu.VMEM((128, H), jnp.float32),
            pltpu.VMEM((128, H), jnp.float32),
            pltpu.VMEM_SHARED((N, H), jnp.float32),
            pltpu.SemaphoreType.DMA,
            pltpu.SemaphoreType.DMA,
            pltpu.SemaphoreType.DMA,
            pltpu.SemaphoreType.DMA,
        ],
    )
    return deg_kernel, agg_kernel


_R = 1000


def _bf16_dot(a, b):
    return jnp.dot(a.astype(jnp.bfloat16), b.astype(jnp.bfloat16),
                   preferred_element_type=jnp.float32)


def _k2_body(x_ref, w_ref, dega_ref, degb_ref, lo_ref, hi_ref):
    xw = _bf16_dot(x_ref[...], w_ref[...])
    xs = xw * lax.rsqrt(dega_ref[...] + degb_ref[...] - 1.0)
    lo_ref[...] = xs[:, :H]
    hi_ref[...] = xs[:, H:]


def _k4_body(lo_ref, hi_ref, dega_ref, degb_ref, x_ref, bg_ref,
             w1_ref, b1_ref, w2_ref, b2_ref, w3_ref, b3_ref,
             x0_ref, x1_ref):
    dinv = lax.rsqrt(dega_ref[...] + degb_ref[...] - 1.0)
    agg = jnp.concatenate([lo_ref[...], hi_ref[...]], axis=1)
    out = agg * dinv + bg_ref[...]
    h = jnp.maximum(out, 0.0) + x_ref[...]
    x0 = jnp.maximum(_bf16_dot(h, w1_ref[...]) + b1_ref[...], 0.0)
    x0_ref[...] = x0
    y = jnp.maximum(_bf16_dot(x0, w2_ref[...]) + b2_ref[...], 0.0)
    x1_ref[...] = _bf16_dot(y, w3_ref[...]) + b3_ref[...]


def kernel(x, edge_index, W_gcn, b_gcn, W1, b1, W2, b2, W3, b3):
    src2 = edge_index[0].astype(jnp.int32).reshape(IDX_ROWS, KC)
    dst2 = edge_index[1].astype(jnp.int32).reshape(IDX_ROWS, KC)

    _deg_kernel, _agg_kernel = _sc_kernels()
    deg = _deg_kernel(dst2)
    deg_a = deg[:N].reshape(N, 1)
    deg_b = deg[N:].reshape(N, 1)

    xs_lo, xs_hi = pl.pallas_call(
        _k2_body,
        grid=(N // _R,),
        in_specs=[
            pl.BlockSpec((_R, C), lambda i: (i, 0)),
            pl.BlockSpec((C, C), lambda i: (0, 0)),
            pl.BlockSpec((_R, 1), lambda i: (i, 0)),
            pl.BlockSpec((_R, 1), lambda i: (i, 0)),
        ],
        out_specs=[
            pl.BlockSpec((_R, H), lambda i: (i, 0)),
            pl.BlockSpec((_R, H), lambda i: (i, 0)),
        ],
        out_shape=[jax.ShapeDtypeStruct((N, H), jnp.float32),
                   jax.ShapeDtypeStruct((N, H), jnp.float32)],
    )(x, W_gcn, deg_a, deg_b)

    agg_lo, agg_hi = _agg_kernel(xs_lo, xs_hi, src2, dst2)

    x0, x1 = pl.pallas_call(
        _k4_body,
        grid=(N // _R,),
        in_specs=[
            pl.BlockSpec((_R, H), lambda i: (i, 0)),
            pl.BlockSpec((_R, H), lambda i: (i, 0)),
            pl.BlockSpec((_R, 1), lambda i: (i, 0)),
            pl.BlockSpec((_R, 1), lambda i: (i, 0)),
            pl.BlockSpec((_R, C), lambda i: (i, 0)),
            pl.BlockSpec((C,), lambda i: (0,)),
            pl.BlockSpec((C, MID), lambda i: (0, 0)),
            pl.BlockSpec((MID,), lambda i: (0,)),
            pl.BlockSpec((MID, MID), lambda i: (0, 0)),
            pl.BlockSpec((MID,), lambda i: (0,)),
            pl.BlockSpec((MID, 1), lambda i: (0, 0)),
            pl.BlockSpec((1,), lambda i: (0,)),
        ],
        out_specs=[
            pl.BlockSpec((_R, MID), lambda i: (i, 0)),
            pl.BlockSpec((_R, 1), lambda i: (i, 0)),
        ],
        out_shape=[jax.ShapeDtypeStruct((N, MID), jnp.float32),
                   jax.ShapeDtypeStruct((N, 1), jnp.float32)],
    )(agg_lo, agg_hi, deg_a, deg_b, x, b_gcn, W1, b1, W2, b2, W3, b3)

    return (x0, x1)


# speedup vs baseline: 1.2769x; 1.2769x over previous
import functools

import jax
import jax.numpy as jnp
from jax import lax
from jax.experimental import pallas as pl
from jax.experimental.pallas import tpu as pltpu
from jax.experimental.pallas import tpu_sc as plsc

N = 10000
E = 160000
C = 256
H = 128
MID = 512

KC = 125
IDX_ROWS = E // KC
TPR = IDX_ROWS // 16
INNER = 8

def _deg_kernel_body(ei3, deg_out, dstv, onesv, stagev, accd):
    c = lax.axis_index("c")
    s = lax.axis_index("s")

    for i in range(8):
        onesv[pl.ds(i * 16, 16)] = jnp.ones((16,), jnp.float32)

    @pl.when(s < 15)
    def _():
        for i in range(5):
            pltpu.sync_copy(onesv, accd.at[pl.ds(s * 640 + i * 128, 128)])

    @pl.when(s == 15)
    def _():
        for i in range(3):
            pltpu.sync_copy(onesv, accd.at[pl.ds(9600 + i * 128, 128)])
        pltpu.sync_copy(onesv.at[pl.ds(0, 16)], accd.at[pl.ds(9984, 16)])

    plsc.subcore_barrier()

    base = (c * 16 + s) * (TPR // 2)

    def outer(j, carry):
        pltpu.sync_copy(ei3.at[1, pl.ds(base + j * INNER, INNER)], dstv)
        for t in range(INNER):
            pltpu.sync_copy(onesv.at[pl.ds(0, KC)],
                            accd.at[dstv.at[t]], add=True)
        return carry

    lax.fori_loop(0, TPR // 2 // INNER, outer, 0)

    plsc.subcore_barrier()

    @pl.when(s < 15)
    def _():
        for i in range(5):
            pltpu.sync_copy(accd.at[pl.ds(s * 640 + i * 128, 128)], stagev)
            pltpu.sync_copy(
                stagev, deg_out.at[pl.ds(c * N + s * 640 + i * 128, 128)])

    @pl.when(s == 15)
    def _():
        for i in range(3):
            pltpu.sync_copy(accd.at[pl.ds(9600 + i * 128, 128)], stagev)
            pltpu.sync_copy(stagev,
                            deg_out.at[pl.ds(c * N + 9600 + i * 128, 128)])
        pltpu.sync_copy(accd.at[pl.ds(9984, 16)], stagev.at[pl.ds(0, 16)])
        pltpu.sync_copy(stagev.at[pl.ds(0, 16)],
                        deg_out.at[pl.ds(c * N + 9984, 16)])


def _agg_kernel_body(xs_lo, xs_hi, ei3, agg_lo, agg_hi,
                     srcv, dstv, rows_a, rows_b, acc, sem_a, sem_b):
    c = lax.axis_index("c")
    s = lax.axis_index("s")
    stage = rows_a

    def run(xs, agg):
        @pl.when(s < 15)
        def _():
            for i in range(5):
                r0 = s * 640 + i * 128
                pltpu.sync_copy(xs.at[pl.ds(r0, 128)], stage)
                pltpu.sync_copy(stage, acc.at[pl.ds(r0, 128)])

        @pl.when(s == 15)
        def _():
            for i in range(3):
                r0 = 9600 + i * 128
                pltpu.sync_copy(xs.at[pl.ds(r0, 128)], stage)
                pltpu.sync_copy(stage, acc.at[pl.ds(r0, 128)])
            pltpu.sync_copy(xs.at[pl.ds(9984, 16)], stage.at[pl.ds(0, 16)])
            pltpu.sync_copy(stage.at[pl.ds(0, 16)], acc.at[pl.ds(9984, 16)])

        plsc.subcore_barrier()
        base = s * TPR
        bufs = (rows_a, rows_b)
        sems = (sem_a, sem_b)
        HALF = TPR // 2

        def gather(t, b):
            return pltpu.async_copy(xs.at[srcv.at[t]],
                                    bufs[b].at[pl.ds(0, KC)], sems[b])

        def scatter(t, b):
            pltpu.sync_copy(bufs[b].at[pl.ds(0, KC)],
                            acc.at[dstv.at[t]], add=True)

        for half in range(2):
            r0 = base + half * HALF
            pltpu.sync_copy(ei3.at[0, pl.ds(r0, HALF)], srcv)
            pltpu.sync_copy(ei3.at[1, pl.ds(r0, HALF)], dstv)
            ga = gather(0, 0)
            gb = gather(1, 1)

            def body(j, carry):
                t0 = j * 2
                ga.wait()
                scatter(t0, 0)

                @pl.when(j < HALF // 2 - 1)
                def _():
                    gather(t0 + 2, 0)

                gb.wait()
                scatter(t0 + 1, 1)

                @pl.when(j < HALF // 2 - 1)
                def _():
                    gather(t0 + 3, 1)

                return carry

            lax.fori_loop(0, HALF // 2, body, 0)

        plsc.subcore_barrier()

        @pl.when(s < 15)
        def _():
            for i in range(5):
                r0 = s * 640 + i * 128
                pltpu.sync_copy(acc.at[pl.ds(r0, 128)], stage)
                pltpu.sync_copy(stage, agg.at[pl.ds(r0, 128)])

        @pl.when(s == 15)
        def _():
            for i in range(3):
                r0 = 9600 + i * 128
                pltpu.sync_copy(acc.at[pl.ds(r0, 128)], stage)
                pltpu.sync_copy(stage, agg.at[pl.ds(r0, 128)])
            pltpu.sync_copy(acc.at[pl.ds(9984, 16)], stage.at[pl.ds(0, 16)])
            pltpu.sync_copy(stage.at[pl.ds(0, 16)], agg.at[pl.ds(9984, 16)])

    @pl.when(c == 0)
    def _():
        run(xs_lo, agg_lo)

    @pl.when(c == 1)
    def _():
        run(xs_hi, agg_hi)


@functools.cache
def _sc_kernels():
    mesh = plsc.VectorSubcoreMesh(core_axis_name="c", subcore_axis_name="s",
                                  num_cores=2, num_subcores=16)
    deg_kernel = pl.kernel(
        _deg_kernel_body,
        out_type=jax.ShapeDtypeStruct((2 * N,), jnp.float32),
        mesh=mesh,
        scratch_types=[
            pltpu.VMEM((INNER, KC), jnp.int32),
            pltpu.VMEM((128,), jnp.float32),
            pltpu.VMEM((128,), jnp.float32),
            pltpu.VMEM_SHARED((N,), jnp.float32),
        ],
    )
    agg_kernel = pl.kernel(
        _agg_kernel_body,
        out_type=[jax.ShapeDtypeStruct((N, H), jnp.float32),
                  jax.ShapeDtypeStruct((N, H), jnp.float32)],
        mesh=mesh,
        scratch_types=[
            pltpu.VMEM((TPR // 2, KC), jnp.int32),
            pltpu.VMEM((TPR // 2, KC), jnp.int32),
            pltpu.VMEM((128, H), jnp.float32),
            pltpu.VMEM((128, H), jnp.float32),
            pltpu.VMEM_SHARED((N, H), jnp.float32),
            pltpu.SemaphoreType.DMA,
            pltpu.SemaphoreType.DMA,
        ],
    )
    return deg_kernel, agg_kernel


_R = 2000


def _bf16_dot(a, b):
    return jnp.dot(a.astype(jnp.bfloat16), b.astype(jnp.bfloat16),
                   preferred_element_type=jnp.float32)


def _k2_body(x_ref, w_ref, deg_ref, lo_ref, hi_ref):
    xw = _bf16_dot(x_ref[...], w_ref[...])
    xs = xw * lax.rsqrt(deg_ref[...])
    lo_ref[...] = xs[:, :H]
    hi_ref[...] = xs[:, H:]


def _k4_body(lo_ref, hi_ref, deg_ref, x_ref, bg_ref,
             w1_ref, b1_ref, w2_ref, b2_ref, w3_ref, b3_ref,
             x0_ref, x1_ref):
    dinv = lax.rsqrt(deg_ref[...])
    agg = jnp.concatenate([lo_ref[...], hi_ref[...]], axis=1)
    out = agg * dinv + bg_ref[...]
    h = jnp.maximum(out, 0.0) + x_ref[...]
    x0 = jnp.maximum(_bf16_dot(h, w1_ref[...]) + b1_ref[...], 0.0)
    x0_ref[...] = x0
    y = jnp.maximum(_bf16_dot(x0, w2_ref[...]) + b2_ref[...], 0.0)
    x1_ref[...] = _bf16_dot(y, w3_ref[...]) + b3_ref[...]


def kernel(x, edge_index, W_gcn, b_gcn, W1, b1, W2, b2, W3, b3):
    ei3 = edge_index.astype(jnp.int32).reshape(2, IDX_ROWS, KC)

    _deg_kernel, _agg_kernel = _sc_kernels()
    deg = _deg_kernel(ei3)
    deg_col = (deg[:N] + deg[N:] - 1.0).reshape(N, 1)

    xs_lo, xs_hi = pl.pallas_call(
        _k2_body,
        grid=(N // _R,),
        in_specs=[
            pl.BlockSpec((_R, C), lambda i: (i, 0)),
            pl.BlockSpec((C, C), lambda i: (0, 0)),
            pl.BlockSpec((_R, 1), lambda i: (i, 0)),
        ],
        out_specs=[
            pl.BlockSpec((_R, H), lambda i: (i, 0)),
            pl.BlockSpec((_R, H), lambda i: (i, 0)),
        ],
        out_shape=[jax.ShapeDtypeStruct((N, H), jnp.float32),
                   jax.ShapeDtypeStruct((N, H), jnp.float32)],
    )(x, W_gcn, deg_col)

    agg_lo, agg_hi = _agg_kernel(xs_lo, xs_hi, ei3)

    x0, x1 = pl.pallas_call(
        _k4_body,
        grid=(N // _R,),
        in_specs=[
            pl.BlockSpec((_R, H), lambda i: (i, 0)),
            pl.BlockSpec((_R, H), lambda i: (i, 0)),
            pl.BlockSpec((_R, 1), lambda i: (i, 0)),
            pl.BlockSpec((_R, C), lambda i: (i, 0)),
            pl.BlockSpec((C,), lambda i: (0,)),
            pl.BlockSpec((C, MID), lambda i: (0, 0)),
            pl.BlockSpec((MID,), lambda i: (0,)),
            pl.BlockSpec((MID, MID), lambda i: (0, 0)),
            pl.BlockSpec((MID,), lambda i: (0,)),
            pl.BlockSpec((MID, 1), lambda i: (0, 0)),
            pl.BlockSpec((1,), lambda i: (0,)),
        ],
        out_specs=[
            pl.BlockSpec((_R, MID), lambda i: (i, 0)),
            pl.BlockSpec((_R, 1), lambda i: (i, 0)),
        ],
        out_shape=[jax.ShapeDtypeStruct((N, MID), jnp.float32),
                   jax.ShapeDtypeStruct((N, 1), jnp.float32)],
    )(agg_lo, agg_hi, deg_col, x, b_gcn, W1, b1, W2, b2, W3, b3)

    return (x0, x1)
